# baseline (speedup 1.0000x reference)
import functools
import jax
import jax.numpy as jnp
from jax import lax
from jax.experimental import pallas as pl
from jax.experimental.pallas import tpu as pltpu
from jax.experimental.pallas import tpu_sc as plsc

_EPS = 1e-5
_NC = 2
_NS = 16
_B = 128
_DEGW = 16
_NB = 6


def _deg_sc(col3, zeros16, ones16):
    ns, k, b = col3.shape
    np_ = zeros16.shape[0]
    rpt = np_ // _NS
    kh = k // 2
    mesh = plsc.VectorSubcoreMesh(core_axis_name="c", subcore_axis_name="s")

    @functools.partial(
        pl.kernel,
        out_type=jax.ShapeDtypeStruct((_NC, np_, _DEGW), jnp.float32),
        mesh=mesh,
        scratch_types=[
            pltpu.VMEM((k, b), jnp.int32),
            pltpu.VMEM((_B, _DEGW), jnp.float32),
            pltpu.VMEM_SHARED((np_, _DEGW), jnp.float32),
        ],
        compiler_params=pltpu.CompilerParams(use_tc_tiling_on_sc=False),
    )
    def deg_kernel(col_hbm, z_hbm, ones_hbm, out_hbm, colb, onesb, deg_s):
        c = lax.axis_index("c")
        s = lax.axis_index("s")
        pltpu.sync_copy(col_hbm.at[s], colb)
        pltpu.sync_copy(ones_hbm, onesb)

        @pl.when(s == 0)
        def _():
            pltpu.sync_copy(z_hbm, deg_s)

        plsc.subcore_barrier()

        def body(j, carry):
            pltpu.sync_copy(onesb, deg_s.at[colb.at[j]], add=True)
            return carry

        lax.fori_loop(c * kh, (c + 1) * kh, body, 0)
        plsc.subcore_barrier()
        off = s * rpt
        pltpu.sync_copy(deg_s.at[pl.ds(off, rpt)],
                        out_hbm.at[c, pl.ds(off, rpt)])

    return deg_kernel(col3, zeros16, ones16)


def _edge_scatter_sc(row3, col3, hs2, zeros_h):
    ns, k, b = row3.shape
    _, np_, dhh = hs2.shape
    rpt = np_ // _NS
    mesh = plsc.VectorSubcoreMesh(core_axis_name="c", subcore_axis_name="s")

    @functools.partial(
        pl.kernel,
        out_type=jax.ShapeDtypeStruct((_NC, np_, dhh), jnp.float32),
        mesh=mesh,
        scratch_types=[
            pltpu.VMEM((k, b), jnp.int32),
            pltpu.VMEM((k, b), jnp.int32),
            pltpu.VMEM((_NB, _B, dhh), jnp.float32),
            pltpu.VMEM_SHARED((np_, dhh), jnp.float32),
        ] + [pltpu.SemaphoreType.DMA] * (_NB + 2),
        compiler_params=pltpu.CompilerParams(use_tc_tiling_on_sc=False),
    )
    def sc_kernel(row_hbm, col_hbm, hs_hbm, z_hbm, out_hbm,
                  rowb, colb, datab, acc_s, *sems):
        gsem = sems[:_NB]
        ssem = sems[_NB:]
        c = lax.axis_index("c")
        s = lax.axis_index("s")
        pltpu.sync_copy(row_hbm.at[s], rowb)
        pltpu.sync_copy(col_hbm.at[s], colb)

        @pl.when(s == 0)
        def _():
            pltpu.sync_copy(z_hbm, acc_s)

        plsc.subcore_barrier()

        lag = _NB - 2
        for u in range(lag):
            pltpu.async_copy(hs_hbm.at[c].at[rowb.at[u]], datab.at[u],
                             gsem[u])

        def body(i, carry):
            j0 = i * _NB
            for du in range(_NB):
                j = j0 + du
                u = du
                w = (du + lag) % _NB
                p = du % 2

                @pl.when(j >= 2)
                def _():
                    pltpu.make_async_copy(datab.at[(u - 2) % _NB],
                                          acc_s.at[colb.at[j - 2]],
                                          ssem[p]).wait()

                @pl.when(j + lag < k)
                def _():
                    pltpu.async_copy(hs_hbm.at[c].at[rowb.at[j + lag]],
                                     datab.at[w], gsem[w])

                pltpu.make_async_copy(hs_hbm.at[c].at[rowb.at[j]],
                                      datab.at[u], gsem[u]).wait()
                pltpu.async_copy(datab.at[u], acc_s.at[colb.at[j]], ssem[p],
                                 add=True)
            return carry

        lax.fori_loop(0, k // _NB, body, 0)
        for i in range(2):
            j = k - 2 + i
            pltpu.make_async_copy(datab.at[j % _NB], acc_s.at[colb.at[j]],
                                  ssem[j % 2]).wait()
        plsc.subcore_barrier()
        off = s * rpt
        pltpu.sync_copy(acc_s.at[pl.ds(off, rpt)],
                        out_hbm.at[c, pl.ds(off, rpt)])

    return sc_kernel(row3, col3, hs2, zeros_h)


def _embed_tc(x_pad, we_t, b_emb, g_emb, be_emb, rm_emb, rv_emb, degp):
    np_, din = x_pad.shape
    dh = we_t.shape[1]
    dhh = dh // 2
    rb = 1264
    grid = np_ // rb

    def body(x_ref, w_ref, b_ref, g_ref, be_ref, rm_ref, rv_ref, deg_ref,
             out_ref):
        h = jnp.dot(x_ref[...], w_ref[...],
                    preferred_element_type=jnp.float32) + b_ref[...]
        h = jnp.maximum(h, 0.0)
        h = (h - rm_ref[...]) * lax.rsqrt(rv_ref[...] + _EPS) * g_ref[...] \
            + be_ref[...]
        deg = deg_ref[0, :, 0:1] + deg_ref[1, :, 0:1] + 1.0
        hs = h * lax.rsqrt(deg)
        out_ref[0] = hs[:, :dhh]
        out_ref[1] = hs[:, dhh:]

    full = lambda shape: pl.BlockSpec(shape, lambda i: (0,) * len(shape))
    return pl.pallas_call(
        body,
        grid=(grid,),
        in_specs=[
            pl.BlockSpec((rb, din), lambda i: (i, 0)),
            full((din, dh)),
            full((1, dh)), full((1, dh)), full((1, dh)),
            full((1, dh)), full((1, dh)),
            pl.BlockSpec((_NC, rb, _DEGW), lambda i: (0, i, 0)),
        ],
        out_specs=pl.BlockSpec((2, rb, dhh), lambda i: (0, i, 0)),
        out_shape=jax.ShapeDtypeStruct((2, np_, dhh), jnp.float32),
    )(x_pad, we_t, b_emb, g_emb, be_emb, rm_emb, rv_emb, degp)


def _head_tc(sp, hs2, degp, wc_t, bc, w1_t, b1, w2_t, b2, w3_t, b3):
    _, np_, dhh = hs2.shape
    dout = w3_t.shape[1]
    rb = 1264
    grid = np_ // rb

    def body(s_ref, hs_ref, deg_ref, wc_ref, bc_ref, w1_ref, b1_ref,
             w2_ref, b2_ref, w3_ref, b3_ref, out_ref):
        deg = deg_ref[0, :, 0:1] + deg_ref[1, :, 0:1] + 1.0
        agg = jnp.concatenate(
            [s_ref[0] + hs_ref[0], s_ref[1] + hs_ref[1]], axis=1)
        agg = agg * lax.rsqrt(deg)
        z = jnp.dot(agg, wc_ref[...],
                    preferred_element_type=jnp.float32) + bc_ref[...]
        z = jnp.maximum(z, 0.0)
        z = jnp.dot(z, w1_ref[...],
                    preferred_element_type=jnp.float32) + b1_ref[...]
        z = jnp.maximum(z, 0.0)
        z = jnp.dot(z, w2_ref[...],
                    preferred_element_type=jnp.float32) + b2_ref[...]
        z = jnp.maximum(z, 0.0)
        z = jnp.dot(z, w3_ref[...],
                    preferred_element_type=jnp.float32) + b3_ref[...]
        out_ref[...] = jax.nn.sigmoid(z)

    full = lambda shape: pl.BlockSpec(shape, lambda i: (0,) * len(shape))
    return pl.pallas_call(
        body,
        grid=(grid,),
        in_specs=[
            pl.BlockSpec((_NC, rb, dhh), lambda i: (0, i, 0)),
            pl.BlockSpec((_NC, rb, dhh), lambda i: (0, i, 0)),
            pl.BlockSpec((_NC, rb, _DEGW), lambda i: (0, i, 0)),
            full(wc_t.shape), full(bc.shape),
            full(w1_t.shape), full(b1.shape),
            full(w2_t.shape), full(b2.shape),
            full(w3_t.shape), full(b3.shape),
        ],
        out_specs=pl.BlockSpec((rb, dout), lambda i: (i, 0)),
        out_shape=jax.ShapeDtypeStruct((np_, dout), jnp.float32),
    )(sp, hs2, degp, wc_t, bc, w1_t, b1, w2_t, b2, w3_t, b3)


def kernel(x, edge_index, W_emb, b_emb, g_emb, be_emb, rm_emb, rv_emb,
           W_conv, b_conv, W1, b1, W2, b2, W3, b3):
    n, din = x.shape
    e = edge_index.shape[1]

    np_ = ((n + 1 + 127) // 128) * 128
    k = -(-e // (_NS * _B))
    k += (-k) % _NB
    ep = _NS * k * _B

    row = edge_index[0]
    col = edge_index[1]
    pad = jnp.full((ep - e,), n, dtype=edge_index.dtype)
    row3 = jnp.concatenate([row, pad]).reshape(_NS, k, _B)
    col3 = jnp.concatenate([col, pad]).reshape(_NS, k, _B)
    x_pad = jnp.pad(x, ((0, np_ - n), (0, 0)))

    zeros16 = jnp.zeros((np_, _DEGW), jnp.float32)
    ones16 = jnp.ones((_B, _DEGW), jnp.float32)
    zeros_h = jnp.zeros((np_, W_emb.shape[0] // 2), jnp.float32)

    row1 = lambda v: v.reshape(1, -1)

    degp = _deg_sc(col3, zeros16, ones16)
    hs2 = _embed_tc(x_pad, W_emb.T, row1(b_emb), row1(g_emb), row1(be_emb),
                    row1(rm_emb), row1(rv_emb), degp)
    sp = _edge_scatter_sc(row3, col3, hs2, zeros_h)
    out = _head_tc(sp, hs2, degp, W_conv.T, row1(b_conv), W1.T, row1(b1),
                   W2.T, row1(b2), W3.T, row1(b3))
    return out[:n]


# speedup vs baseline: 1.5378x; 1.5378x over previous
import functools
import jax
import jax.numpy as jnp
from jax import lax
from jax.experimental import pallas as pl
from jax.experimental.pallas import tpu as pltpu
from jax.experimental.pallas import tpu_sc as plsc

_EPS = 1e-5
_NC = 2
_NS = 16
_B = 128
_DEGW = 16


def _deg_sc(col3, zeros16, ones16):
    ns, k, b = col3.shape
    np_ = zeros16.shape[0]
    rpt = np_ // _NS
    kh = k // 2
    mesh = plsc.VectorSubcoreMesh(core_axis_name="c", subcore_axis_name="s")

    @functools.partial(
        pl.kernel,
        out_type=jax.ShapeDtypeStruct((_NC, np_, _DEGW), jnp.float32),
        mesh=mesh,
        scratch_types=[
            pltpu.VMEM((k, b), jnp.int32),
            pltpu.VMEM((_B, _DEGW), jnp.float32),
            pltpu.VMEM_SHARED((np_, _DEGW), jnp.float32),
        ],
        compiler_params=pltpu.CompilerParams(use_tc_tiling_on_sc=False),
    )
    def deg_kernel(col_hbm, z_hbm, ones_hbm, out_hbm, colb, onesb, deg_s):
        c = lax.axis_index("c")
        s = lax.axis_index("s")
        pltpu.sync_copy(col_hbm.at[s], colb)
        pltpu.sync_copy(ones_hbm, onesb)

        @pl.when(s == 0)
        def _():
            pltpu.sync_copy(z_hbm, deg_s)

        plsc.subcore_barrier()

        def body(j, carry):
            pltpu.sync_copy(onesb, deg_s.at[colb.at[j]], add=True)
            return carry

        lax.fori_loop(c * kh, (c + 1) * kh, body, 0)
        plsc.subcore_barrier()
        off = s * rpt
        pltpu.sync_copy(deg_s.at[pl.ds(off, rpt)],
                        out_hbm.at[c, pl.ds(off, rpt)])

    return deg_kernel(col3, zeros16, ones16)


def _edge_scatter_sc(row3, col3, hs2, zeros_h):
    ns, k, b = row3.shape
    _, np_, dhh = hs2.shape
    rpt = np_ // _NS
    mesh = plsc.VectorSubcoreMesh(core_axis_name="c", subcore_axis_name="s")

    @functools.partial(
        pl.kernel,
        out_type=jax.ShapeDtypeStruct((_NC, np_, dhh), jnp.float32),
        mesh=mesh,
        scratch_types=[
            pltpu.VMEM((k, b), jnp.int32),
            pltpu.VMEM((k, b), jnp.int32),
            pltpu.VMEM((4, _B, dhh), jnp.float32),
            pltpu.VMEM_SHARED((np_, dhh), jnp.float32),
            pltpu.SemaphoreType.DMA,
            pltpu.SemaphoreType.DMA,
            pltpu.SemaphoreType.DMA,
            pltpu.SemaphoreType.DMA,
        ],
        compiler_params=pltpu.CompilerParams(use_tc_tiling_on_sc=False),
    )
    def sc_kernel(row_hbm, col_hbm, hs_hbm, z_hbm, out_hbm,
                  rowb, colb, datab, acc_s, *gsem):
        c = lax.axis_index("c")
        s = lax.axis_index("s")
        pltpu.sync_copy(row_hbm.at[s], rowb)
        pltpu.sync_copy(col_hbm.at[s], colb)

        @pl.when(s == 0)
        def _():
            pltpu.sync_copy(z_hbm, acc_s)

        plsc.subcore_barrier()

        for u in range(3):
            pltpu.async_copy(hs_hbm.at[c].at[rowb.at[u]], datab.at[u],
                             gsem[u])

        def body(i, carry):
            j0 = i * 4
            for du in range(4):
                j = j0 + du
                u = du
                w = (du + 3) % 4
                pltpu.make_async_copy(hs_hbm.at[c].at[rowb.at[j]],
                                      datab.at[u], gsem[u]).wait()
                pltpu.sync_copy(datab.at[u], acc_s.at[colb.at[j]], add=True)

                @pl.when(j + 3 < k)
                def _():
                    pltpu.async_copy(hs_hbm.at[c].at[rowb.at[j + 3]],
                                     datab.at[w], gsem[w])
            return carry

        lax.fori_loop(0, k // 4, body, 0)
        plsc.subcore_barrier()
        off = s * rpt
        pltpu.sync_copy(acc_s.at[pl.ds(off, rpt)],
                        out_hbm.at[c, pl.ds(off, rpt)])

    return sc_kernel(row3, col3, hs2, zeros_h)


def _embed_tc(x_pad, we_t, b_emb, g_emb, be_emb, rm_emb, rv_emb, degp):
    np_, din = x_pad.shape
    dh = we_t.shape[1]
    dhh = dh // 2
    rb = 1264
    grid = np_ // rb

    def body(x_ref, w_ref, b_ref, g_ref, be_ref, rm_ref, rv_ref, deg_ref,
             out_ref):
        h = jnp.dot(x_ref[...], w_ref[...],
                    preferred_element_type=jnp.float32) + b_ref[...]
        h = jnp.maximum(h, 0.0)
        h = (h - rm_ref[...]) * lax.rsqrt(rv_ref[...] + _EPS) * g_ref[...] \
            + be_ref[...]
        deg = deg_ref[0, :, 0:1] + deg_ref[1, :, 0:1] + 1.0
        hs = h * lax.rsqrt(deg)
        out_ref[0] = hs[:, :dhh]
        out_ref[1] = hs[:, dhh:]

    full = lambda shape: pl.BlockSpec(shape, lambda i: (0,) * len(shape))
    return pl.pallas_call(
        body,
        grid=(grid,),
        in_specs=[
            pl.BlockSpec((rb, din), lambda i: (i, 0)),
            full((din, dh)),
            full((1, dh)), full((1, dh)), full((1, dh)),
            full((1, dh)), full((1, dh)),
            pl.BlockSpec((_NC, rb, _DEGW), lambda i: (0, i, 0)),
        ],
        out_specs=pl.BlockSpec((2, rb, dhh), lambda i: (0, i, 0)),
        out_shape=jax.ShapeDtypeStruct((2, np_, dhh), jnp.float32),
    )(x_pad, we_t, b_emb, g_emb, be_emb, rm_emb, rv_emb, degp)


def _head_tc(sp, hs2, degp, wc_t, bc, w1_t, b1, w2_t, b2, w3_t, b3):
    _, np_, dhh = hs2.shape
    dout = w3_t.shape[1]
    rb = 1264
    grid = np_ // rb

    def body(s_ref, hs_ref, deg_ref, wc_ref, bc_ref, w1_ref, b1_ref,
             w2_ref, b2_ref, w3_ref, b3_ref, out_ref):
        deg = deg_ref[0, :, 0:1] + deg_ref[1, :, 0:1] + 1.0
        agg = jnp.concatenate(
            [s_ref[0] + hs_ref[0], s_ref[1] + hs_ref[1]], axis=1)
        agg = agg * lax.rsqrt(deg)
        z = jnp.dot(agg, wc_ref[...],
                    preferred_element_type=jnp.float32) + bc_ref[...]
        z = jnp.maximum(z, 0.0)
        z = jnp.dot(z, w1_ref[...],
                    preferred_element_type=jnp.float32) + b1_ref[...]
        z = jnp.maximum(z, 0.0)
        z = jnp.dot(z, w2_ref[...],
                    preferred_element_type=jnp.float32) + b2_ref[...]
        z = jnp.maximum(z, 0.0)
        z = jnp.dot(z, w3_ref[...],
                    preferred_element_type=jnp.float32) + b3_ref[...]
        out_ref[...] = jax.nn.sigmoid(z)

    full = lambda shape: pl.BlockSpec(shape, lambda i: (0,) * len(shape))
    return pl.pallas_call(
        body,
        grid=(grid,),
        in_specs=[
            pl.BlockSpec((_NC, rb, dhh), lambda i: (0, i, 0)),
            pl.BlockSpec((_NC, rb, dhh), lambda i: (0, i, 0)),
            pl.BlockSpec((_NC, rb, _DEGW), lambda i: (0, i, 0)),
            full(wc_t.shape), full(bc.shape),
            full(w1_t.shape), full(b1.shape),
            full(w2_t.shape), full(b2.shape),
            full(w3_t.shape), full(b3.shape),
        ],
        out_specs=pl.BlockSpec((rb, dout), lambda i: (i, 0)),
        out_shape=jax.ShapeDtypeStruct((np_, dout), jnp.float32),
    )(sp, hs2, degp, wc_t, bc, w1_t, b1, w2_t, b2, w3_t, b3)


def kernel(x, edge_index, W_emb, b_emb, g_emb, be_emb, rm_emb, rv_emb,
           W_conv, b_conv, W1, b1, W2, b2, W3, b3):
    n, din = x.shape
    e = edge_index.shape[1]

    np_ = ((n + 1 + 127) // 128) * 128
    k = -(-e // (_NS * _B))
    k += (-k) % 4
    ep = _NS * k * _B

    row = edge_index[0]
    col = edge_index[1]
    pad = jnp.full((ep - e,), n, dtype=edge_index.dtype)
    row3 = jnp.concatenate([row, pad]).reshape(_NS, k, _B)
    col3 = jnp.concatenate([col, pad]).reshape(_NS, k, _B)
    x_pad = jnp.pad(x, ((0, np_ - n), (0, 0)))

    zeros16 = jnp.zeros((np_, _DEGW), jnp.float32)
    ones16 = jnp.ones((_B, _DEGW), jnp.float32)
    zeros_h = jnp.zeros((np_, W_emb.shape[0] // 2), jnp.float32)

    row1 = lambda v: v.reshape(1, -1)

    degp = _deg_sc(col3, zeros16, ones16)
    hs2 = _embed_tc(x_pad, W_emb.T, row1(b_emb), row1(g_emb), row1(be_emb),
                    row1(rm_emb), row1(rv_emb), degp)
    sp = _edge_scatter_sc(row3, col3, hs2, zeros_h)
    out = _head_tc(sp, hs2, degp, W_conv.T, row1(b_conv), W1.T, row1(b1),
                   W2.T, row1(b2), W3.T, row1(b3))
    return out[:n]


# speedup vs baseline: 1.9785x; 1.2866x over previous
import functools
import jax
import jax.numpy as jnp
from jax import lax
from jax.experimental import pallas as pl
from jax.experimental.pallas import tpu as pltpu
from jax.experimental.pallas import tpu_sc as plsc

_EPS = 1e-5
_NC = 2
_NS = 16
_B = 128
_DEGW = 16
_RK = 16


def _deg_sc(col3, zeros16, ones16):
    ns, k, b = col3.shape
    np_ = zeros16.shape[0]
    rpt = np_ // _NS
    kh = k // 2
    mesh = plsc.VectorSubcoreMesh(core_axis_name="c", subcore_axis_name="s")

    @functools.partial(
        pl.kernel,
        out_type=jax.ShapeDtypeStruct((_NC, np_, _DEGW), jnp.float32),
        mesh=mesh,
        scratch_types=[
            pltpu.VMEM((k, b), jnp.int32),
            pltpu.VMEM((_B, _DEGW), jnp.float32),
            pltpu.VMEM_SHARED((np_, _DEGW), jnp.float32),
        ],
        compiler_params=pltpu.CompilerParams(use_tc_tiling_on_sc=False),
    )
    def deg_kernel(col_hbm, z_hbm, ones_hbm, out_hbm, colb, onesb, deg_s):
        c = lax.axis_index("c")
        s = lax.axis_index("s")
        pltpu.sync_copy(col_hbm.at[s], colb)
        pltpu.sync_copy(ones_hbm, onesb)

        @pl.when(s == 0)
        def _():
            pltpu.sync_copy(z_hbm, deg_s)

        plsc.subcore_barrier()

        def body(j, carry):
            pltpu.sync_copy(onesb, deg_s.at[colb.at[j]], add=True)
            return carry

        lax.fori_loop(c * kh, (c + 1) * kh, body, 0)
        plsc.subcore_barrier()
        off = s * rpt
        pltpu.sync_copy(deg_s.at[pl.ds(off, rpt)],
                        out_hbm.at[c, pl.ds(off, rpt)])

    return deg_kernel(col3, zeros16, ones16)


def _edge_scatter_sc(row3, col3, hs2, zeros_h):
    ns, k, b = row3.shape
    _, np_, dhh = hs2.shape
    rpt = np_ // _NS
    nblk = k // _RK
    mesh = plsc.VectorSubcoreMesh(core_axis_name="c", subcore_axis_name="s")

    @functools.partial(
        pl.kernel,
        out_type=jax.ShapeDtypeStruct((_NC, np_, dhh), jnp.float32),
        mesh=mesh,
        scratch_types=[
            pltpu.VMEM((2, _RK, b), jnp.int32),
            pltpu.VMEM((k, b), jnp.int32),
            pltpu.VMEM((2, _B, dhh), jnp.float32),
            pltpu.VMEM_SHARED((np_, dhh), jnp.float32),
            pltpu.VMEM_SHARED((np_, dhh), jnp.float32),
            pltpu.SemaphoreType.DMA,
            pltpu.SemaphoreType.DMA,
            pltpu.SemaphoreType.DMA,
            pltpu.SemaphoreType.DMA,
        ],
        compiler_params=pltpu.CompilerParams(use_tc_tiling_on_sc=False),
    )
    def sc_kernel(row_hbm, col_hbm, hs_hbm, z_hbm, out_hbm,
                  rowb, colb, datab, hs_s, acc_s,
                  gsem0, gsem1, isem0, isem1):
        c = lax.axis_index("c")
        s = lax.axis_index("s")
        off = s * rpt
        pltpu.sync_copy(col_hbm.at[s], colb)
        pltpu.sync_copy(hs_hbm.at[c, pl.ds(off, rpt)],
                        hs_s.at[pl.ds(off, rpt)])

        @pl.when(s == 0)
        def _():
            pltpu.sync_copy(z_hbm, acc_s)

        plsc.subcore_barrier()

        pltpu.async_copy(row_hbm.at[s, pl.ds(0, _RK)], rowb.at[0], isem0)
        pltpu.async_copy(row_hbm.at[s, pl.ds(_RK, _RK)], rowb.at[1], isem1)

        def block(base, r, isem):
            pltpu.async_copy(hs_s.at[rowb.at[r, 0]], datab.at[0], gsem0)
            for jj in range(_RK):
                u = jj % 2
                gsem = gsem0 if u == 0 else gsem1
                osem = gsem1 if u == 0 else gsem0
                if jj + 1 < _RK:
                    pltpu.async_copy(hs_s.at[rowb.at[r, jj + 1]],
                                     datab.at[1 - u], osem)
                pltpu.make_async_copy(hs_s.at[rowb.at[r, jj]], datab.at[u],
                                      gsem).wait()
                pltpu.sync_copy(datab.at[u], acc_s.at[colb.at[base + jj]],
                                add=True)

        def body(i, carry):
            b0 = i * 2
            base0 = b0 * _RK
            pltpu.make_async_copy(row_hbm.at[s, pl.ds(base0, _RK)],
                                  rowb.at[0], isem0).wait()
            block(base0, 0, isem0)

            @pl.when(b0 + 2 < nblk)
            def _():
                pltpu.async_copy(row_hbm.at[s, pl.ds(base0 + 2 * _RK, _RK)],
                                 rowb.at[0], isem0)

            base1 = base0 + _RK
            pltpu.make_async_copy(row_hbm.at[s, pl.ds(base1, _RK)],
                                  rowb.at[1], isem1).wait()
            block(base1, 1, isem1)

            @pl.when(b0 + 3 < nblk)
            def _():
                pltpu.async_copy(row_hbm.at[s, pl.ds(base1 + 2 * _RK, _RK)],
                                 rowb.at[1], isem1)

            return carry

        lax.fori_loop(0, nblk // 2, body, 0)
        plsc.subcore_barrier()
        pltpu.sync_copy(acc_s.at[pl.ds(off, rpt)],
                        out_hbm.at[c, pl.ds(off, rpt)])

    return sc_kernel(row3, col3, hs2, zeros_h)


def _embed_tc(x_pad, we_t, b_emb, g_emb, be_emb, rm_emb, rv_emb, degp):
    np_, din = x_pad.shape
    dh = we_t.shape[1]
    dhh = dh // 2
    rb = 1264
    grid = np_ // rb

    def body(x_ref, w_ref, b_ref, g_ref, be_ref, rm_ref, rv_ref, deg_ref,
             out_ref):
        h = jnp.dot(x_ref[...], w_ref[...],
                    preferred_element_type=jnp.float32) + b_ref[...]
        h = jnp.maximum(h, 0.0)
        h = (h - rm_ref[...]) * lax.rsqrt(rv_ref[...] + _EPS) * g_ref[...] \
            + be_ref[...]
        deg = deg_ref[0, :, 0:1] + deg_ref[1, :, 0:1] + 1.0
        hs = h * lax.rsqrt(deg)
        out_ref[0] = hs[:, :dhh]
        out_ref[1] = hs[:, dhh:]

    full = lambda shape: pl.BlockSpec(shape, lambda i: (0,) * len(shape))
    return pl.pallas_call(
        body,
        grid=(grid,),
        in_specs=[
            pl.BlockSpec((rb, din), lambda i: (i, 0)),
            full((din, dh)),
            full((1, dh)), full((1, dh)), full((1, dh)),
            full((1, dh)), full((1, dh)),
            pl.BlockSpec((_NC, rb, _DEGW), lambda i: (0, i, 0)),
        ],
        out_specs=pl.BlockSpec((2, rb, dhh), lambda i: (0, i, 0)),
        out_shape=jax.ShapeDtypeStruct((2, np_, dhh), jnp.float32),
    )(x_pad, we_t, b_emb, g_emb, be_emb, rm_emb, rv_emb, degp)


def _head_tc(sp, hs2, degp, wc_t, bc, w1_t, b1, w2_t, b2, w3_t, b3):
    _, np_, dhh = hs2.shape
    dout = w3_t.shape[1]
    rb = 1264
    grid = np_ // rb

    def body(s_ref, hs_ref, deg_ref, wc_ref, bc_ref, w1_ref, b1_ref,
             w2_ref, b2_ref, w3_ref, b3_ref, out_ref):
        deg = deg_ref[0, :, 0:1] + deg_ref[1, :, 0:1] + 1.0
        agg = jnp.concatenate(
            [s_ref[0] + hs_ref[0], s_ref[1] + hs_ref[1]], axis=1)
        agg = agg * lax.rsqrt(deg)
        z = jnp.dot(agg, wc_ref[...],
                    preferred_element_type=jnp.float32) + bc_ref[...]
        z = jnp.maximum(z, 0.0)
        z = jnp.dot(z, w1_ref[...],
                    preferred_element_type=jnp.float32) + b1_ref[...]
        z = jnp.maximum(z, 0.0)
        z = jnp.dot(z, w2_ref[...],
                    preferred_element_type=jnp.float32) + b2_ref[...]
        z = jnp.maximum(z, 0.0)
        z = jnp.dot(z, w3_ref[...],
                    preferred_element_type=jnp.float32) + b3_ref[...]
        out_ref[...] = jax.nn.sigmoid(z)

    full = lambda shape: pl.BlockSpec(shape, lambda i: (0,) * len(shape))
    return pl.pallas_call(
        body,
        grid=(grid,),
        in_specs=[
            pl.BlockSpec((_NC, rb, dhh), lambda i: (0, i, 0)),
            pl.BlockSpec((_NC, rb, dhh), lambda i: (0, i, 0)),
            pl.BlockSpec((_NC, rb, _DEGW), lambda i: (0, i, 0)),
            full(wc_t.shape), full(bc.shape),
            full(w1_t.shape), full(b1.shape),
            full(w2_t.shape), full(b2.shape),
            full(w3_t.shape), full(b3.shape),
        ],
        out_specs=pl.BlockSpec((rb, dout), lambda i: (i, 0)),
        out_shape=jax.ShapeDtypeStruct((np_, dout), jnp.float32),
    )(sp, hs2, degp, wc_t, bc, w1_t, b1, w2_t, b2, w3_t, b3)


def kernel(x, edge_index, W_emb, b_emb, g_emb, be_emb, rm_emb, rv_emb,
           W_conv, b_conv, W1, b1, W2, b2, W3, b3):
    n, din = x.shape
    e = edge_index.shape[1]

    np_ = ((n + 1 + 127) // 128) * 128
    k = -(-e // (_NS * _B))
    k += (-k) % (2 * _RK)
    ep = _NS * k * _B

    row = edge_index[0]
    col = edge_index[1]
    pad = jnp.full((ep - e,), n, dtype=edge_index.dtype)
    row3 = jnp.concatenate([row, pad]).reshape(_NS, k, _B)
    col3 = jnp.concatenate([col, pad]).reshape(_NS, k, _B)
    x_pad = jnp.pad(x, ((0, np_ - n), (0, 0)))

    zeros16 = jnp.zeros((np_, _DEGW), jnp.float32)
    ones16 = jnp.ones((_B, _DEGW), jnp.float32)
    zeros_h = jnp.zeros((np_, W_emb.shape[0] // 2), jnp.float32)

    row1 = lambda v: v.reshape(1, -1)

    degp = _deg_sc(col3, zeros16, ones16)
    hs2 = _embed_tc(x_pad, W_emb.T, row1(b_emb), row1(g_emb), row1(be_emb),
                    row1(rm_emb), row1(rv_emb), degp)
    sp = _edge_scatter_sc(row3, col3, hs2, zeros_h)
    out = _head_tc(sp, hs2, degp, W_conv.T, row1(b_conv), W1.T, row1(b1),
                   W2.T, row1(b2), W3.T, row1(b3))
    return out[:n]
